# baseline (speedup 1.0000x reference)
import functools

import jax
import jax.numpy as jnp
from jax import lax
from jax.experimental import pallas as pl
from jax.experimental.pallas import tpu as pltpu
from jax.experimental.pallas import tpu_sc as plsc

NC = 2
NS = 16
NW = NC * NS
LANES = 16
NBUF = 4
BPW = 128


@functools.lru_cache(maxsize=None)
def _build(batch, seq_len, vocab, d):
    assert batch == NW * BPW
    assert d % LANES == 0
    assert seq_len % NBUF == 0
    groups = d // LANES

    mesh = plsc.VectorSubcoreMesh(core_axis_name="c", subcore_axis_name="s")

    @functools.partial(
        pl.kernel,
        mesh=mesh,
        out_type=jax.ShapeDtypeStruct((seq_len, d // 8, NW, 8 * BPW),
                                      jnp.float32),
        compiler_params=pltpu.CompilerParams(use_tc_tiling_on_sc=False,
                                             needs_layout_passes=False),
        scratch_types=[
            pltpu.VMEM((seq_len, BPW), jnp.int32),
            pltpu.VMEM((seq_len, d), jnp.float32),
            pltpu.VMEM((NBUF, BPW, d), jnp.float32),
            pltpu.VMEM((2, d * BPW), jnp.float32),
            [pltpu.SemaphoreType.DMA] * NBUF,
            [pltpu.SemaphoreType.DMA] * 2,
        ],
    )
    def embed(table_hbm, idx_hbm, pos_hbm, out_hbm, idx_v, pos_v,
              gbuf, tbuf, gsems, wsems):
        wid = lax.axis_index("s") * NC + lax.axis_index("c")
        col = wid * BPW

        pltpu.sync_copy(idx_hbm.at[:, pl.ds(col, BPW)], idx_v)
        pltpu.sync_copy(pos_hbm, pos_v)

        for b in range(NBUF):
            pltpu.async_copy(table_hbm.at[idx_v.at[b]], gbuf.at[b], gsems[b])

        lane = lax.iota(jnp.int32, LANES)

        @pl.loop(0, seq_len, step=2 * NBUF)
        def s_loop(s0):
            for b2 in range(2 * NBUF):
                s = s0 + b2
                g = b2 % NBUF
                t = b2 % 2
                src = gbuf.at[g]
                dst = tbuf.at[t]

                pltpu.make_async_copy(table_hbm.at[idx_v.at[s]], src,
                                      gsems[g]).wait()

                @pl.when(s >= 2)
                def _():
                    for fb in range(d // 8):
                        pltpu.make_async_copy(
                            tbuf.at[t].at[pl.ds(fb * 8 * BPW, 8 * BPW)],
                            out_hbm.at[s - 2, fb, wid], wsems[t]).wait()

                pvs = [pos_v[s, pl.ds(c * LANES, LANES)]
                       for c in range(groups)]
                fbase = [(c * LANES + lane) * BPW for c in range(groups)]

                @pl.loop(0, BPW, unroll=8)
                def t_loop(r):
                    for c in range(groups):
                        v = src[r, pl.ds(c * LANES, LANES)] + pvs[c]
                        plsc.store_scatter(dst, [fbase[c] + r], v)

                for fb in range(d // 8):
                    pltpu.async_copy(
                        tbuf.at[t].at[pl.ds(fb * 8 * BPW, 8 * BPW)],
                        out_hbm.at[s, fb, wid], wsems[t])

                @pl.when(s + NBUF < seq_len)
                def _():
                    pltpu.async_copy(table_hbm.at[idx_v.at[s + NBUF]],
                                     gbuf.at[g], gsems[g])

        for t in range(2):
            s = seq_len - 2 + t
            for fb in range(d // 8):
                pltpu.make_async_copy(
                    tbuf.at[t].at[pl.ds(fb * 8 * BPW, 8 * BPW)],
                    out_hbm.at[s, fb, wid], wsems[t]).wait()

    return embed


def kernel(inputs, word_table, pos_table):
    batch, seq_len = inputs.shape
    vocab, d = word_table.shape

    embed = _build(batch, seq_len, vocab, d)
    idx_t = inputs.T
    out4 = embed(word_table, idx_t, pos_table)
    out5 = out4.reshape(seq_len, d // 8, NW, 8, BPW)
    return jnp.transpose(out5, (2, 4, 0, 1, 3)).reshape(batch, seq_len, d)


# speedup vs baseline: 1.1864x; 1.1864x over previous
import functools

import jax
import jax.numpy as jnp
from jax import lax
from jax.experimental import pallas as pl
from jax.experimental.pallas import tpu as pltpu
from jax.experimental.pallas import tpu_sc as plsc

NC = 2
NS = 16
NW = NC * NS
LANES = 16
NBUF = 4
BPW = 128


@functools.lru_cache(maxsize=None)
def _build(batch, seq_len, vocab, d):
    assert batch == NW * BPW
    assert d % LANES == 0
    assert seq_len % NBUF == 0
    groups = d // LANES

    mesh = plsc.VectorSubcoreMesh(core_axis_name="c", subcore_axis_name="s")

    @functools.partial(
        pl.kernel,
        mesh=mesh,
        out_type=jax.ShapeDtypeStruct((seq_len, d // 8, NW, 8 * BPW),
                                      jnp.float32),
        compiler_params=pltpu.CompilerParams(use_tc_tiling_on_sc=False,
                                             needs_layout_passes=False),
        scratch_types=[
            pltpu.VMEM((seq_len, BPW), jnp.int32),
            pltpu.VMEM((seq_len, d), jnp.float32),
            pltpu.VMEM((NBUF, BPW, d), jnp.float32),
            pltpu.VMEM((d * (BPW + 1),), jnp.float32),
            pltpu.VMEM((2, d * BPW), jnp.float32),
            [pltpu.SemaphoreType.DMA] * NBUF,
            [pltpu.SemaphoreType.DMA] * 2,
        ],
    )
    def embed(table_hbm, idx_hbm, pos_hbm, out_hbm, idx_v, pos_v,
              gbuf, pbuf, tbuf, gsems, wsems):
        wid = lax.axis_index("s") * NC + lax.axis_index("c")
        col = wid * BPW

        pltpu.sync_copy(idx_hbm.at[:, pl.ds(col, BPW)], idx_v)
        pltpu.sync_copy(pos_hbm, pos_v)

        for b in range(NBUF):
            pltpu.async_copy(table_hbm.at[idx_v.at[b]], gbuf.at[b], gsems[b])

        lane = lax.iota(jnp.int32, LANES)

        @pl.loop(0, seq_len, step=2 * NBUF)
        def s_loop(s0):
            for b2 in range(2 * NBUF):
                s = s0 + b2
                g = b2 % NBUF
                t = b2 % 2
                src = gbuf.at[g]
                dst = tbuf.at[t]

                pltpu.make_async_copy(table_hbm.at[idx_v.at[s]], src,
                                      gsems[g]).wait()

                @pl.when(s >= 2)
                def _():
                    for fb in range(d // 8):
                        pltpu.make_async_copy(
                            tbuf.at[t].at[pl.ds(fb * 8 * BPW, 8 * BPW)],
                            out_hbm.at[s - 2, fb, wid], wsems[t]).wait()

                pvs = [pos_v[s, pl.ds(c * LANES, LANES)]
                       for c in range(groups)]
                fbase = [(c * LANES + lane) * (BPW + 1) for c in range(groups)]

                @pl.loop(0, BPW, unroll=8)
                def t_loop(r):
                    for c in range(groups):
                        v = src[r, pl.ds(c * LANES, LANES)] + pvs[c]
                        plsc.store_scatter(pbuf, [fbase[c] + r], v)

                @pl.loop(0, d, unroll=4)
                def c_loop(f):
                    for c in range(BPW // LANES):
                        dst[pl.ds(f * BPW + c * LANES, LANES)] = (
                            pbuf[pl.ds(f * (BPW + 1) + c * LANES, LANES)])

                for fb in range(d // 8):
                    pltpu.async_copy(
                        tbuf.at[t].at[pl.ds(fb * 8 * BPW, 8 * BPW)],
                        out_hbm.at[s, fb, wid], wsems[t])

                @pl.when(s + NBUF < seq_len)
                def _():
                    pltpu.async_copy(table_hbm.at[idx_v.at[s + NBUF]],
                                     gbuf.at[g], gsems[g])

        for t in range(2):
            s = seq_len - 2 + t
            for fb in range(d // 8):
                pltpu.make_async_copy(
                    tbuf.at[t].at[pl.ds(fb * 8 * BPW, 8 * BPW)],
                    out_hbm.at[s, fb, wid], wsems[t]).wait()

    return embed


def kernel(inputs, word_table, pos_table):
    batch, seq_len = inputs.shape
    vocab, d = word_table.shape

    embed = _build(batch, seq_len, vocab, d)
    idx_t = inputs.T
    out4 = embed(word_table, idx_t, pos_table)
    out5 = out4.reshape(seq_len, d // 8, NW, 8, BPW)
    return jnp.transpose(out5, (2, 4, 0, 1, 3)).reshape(batch, seq_len, d)


# speedup vs baseline: 2.2978x; 1.9367x over previous
import functools

import jax
import jax.numpy as jnp
from jax import lax
from jax.experimental import pallas as pl
from jax.experimental.pallas import tpu as pltpu
from jax.experimental.pallas import tpu_sc as plsc

NC = 2
NS = 16
NW = NC * NS
LANES = 16
NBUF = 4
BPW = 128


@functools.lru_cache(maxsize=None)
def _build(batch, seq_len, vocab, d):
    assert batch == NW * BPW
    assert d % LANES == 0
    assert seq_len % NBUF == 0
    groups = d // LANES

    mesh = plsc.VectorSubcoreMesh(core_axis_name="c", subcore_axis_name="s")

    @functools.partial(
        pl.kernel,
        mesh=mesh,
        out_type=jax.ShapeDtypeStruct((seq_len, d // 8, NW, 8 * BPW),
                                      jnp.float32),
        compiler_params=pltpu.CompilerParams(use_tc_tiling_on_sc=False,
                                             needs_layout_passes=False),
        scratch_types=[
            pltpu.VMEM((seq_len, BPW), jnp.int32),
            pltpu.VMEM((seq_len, d), jnp.float32),
            pltpu.VMEM((NBUF, BPW, d), jnp.float32),
            pltpu.VMEM((d * (BPW + 1),), jnp.float32),
            pltpu.VMEM((2, d * BPW), jnp.float32),
            [pltpu.SemaphoreType.DMA] * NBUF,
            [pltpu.SemaphoreType.DMA] * 2,
        ],
    )
    def embed(table_hbm, idx_hbm, pos_hbm, out_hbm, idx_v, pos_v,
              gbuf, pbuf, tbuf, gsems, wsems):
        wid = lax.axis_index("s") * NC + lax.axis_index("c")
        col = wid * BPW

        pltpu.sync_copy(idx_hbm.at[:, pl.ds(col, BPW)], idx_v)
        pltpu.sync_copy(pos_hbm, pos_v)

        for b in range(NBUF):
            pltpu.async_copy(table_hbm.at[idx_v.at[b]], gbuf.at[b], gsems[b])

        lane = lax.iota(jnp.int32, LANES)

        @pl.loop(0, seq_len, step=2 * NBUF)
        def s_loop(s0):
            for b2 in range(2 * NBUF):
                s = s0 + b2
                g = b2 % NBUF
                t = b2 % 2
                src = gbuf.at[g]
                dst = tbuf.at[t]

                pltpu.make_async_copy(table_hbm.at[idx_v.at[s]], src,
                                      gsems[g]).wait()

                @pl.when(s >= 2)
                def _():
                    for fb in range(d // 8):
                        pltpu.make_async_copy(
                            tbuf.at[t].at[pl.ds(fb * 8 * BPW, 8 * BPW)],
                            out_hbm.at[s - 2, fb, wid], wsems[t]).wait()

                pvs = [pos_v[s, pl.ds(c * LANES, LANES)]
                       for c in range(groups)]
                fbase = [(c * LANES + lane) * (BPW + 1) for c in range(groups)]

                @functools.partial(plsc.parallel_loop, 0, BPW, unroll=8)
                def t_loop(r):
                    for c in range(groups):
                        v = src[r, pl.ds(c * LANES, LANES)] + pvs[c]
                        plsc.store_scatter(pbuf, [fbase[c] + r], v)

                @functools.partial(plsc.parallel_loop, 0, d, unroll=4)
                def c_loop(f):
                    for c in range(BPW // LANES):
                        dst[pl.ds(f * BPW + c * LANES, LANES)] = (
                            pbuf[pl.ds(f * (BPW + 1) + c * LANES, LANES)])

                for fb in range(d // 8):
                    pltpu.async_copy(
                        tbuf.at[t].at[pl.ds(fb * 8 * BPW, 8 * BPW)],
                        out_hbm.at[s, fb, wid], wsems[t])

                @pl.when(s + NBUF < seq_len)
                def _():
                    pltpu.async_copy(table_hbm.at[idx_v.at[s + NBUF]],
                                     gbuf.at[g], gsems[g])

        for t in range(2):
            s = seq_len - 2 + t
            for fb in range(d // 8):
                pltpu.make_async_copy(
                    tbuf.at[t].at[pl.ds(fb * 8 * BPW, 8 * BPW)],
                    out_hbm.at[s, fb, wid], wsems[t]).wait()

    return embed


def kernel(inputs, word_table, pos_table):
    batch, seq_len = inputs.shape
    vocab, d = word_table.shape

    embed = _build(batch, seq_len, vocab, d)
    idx_t = inputs.T
    out4 = embed(word_table, idx_t, pos_table)
    out5 = out4.reshape(seq_len, d // 8, NW, 8, BPW)
    return jnp.transpose(out5, (2, 4, 0, 1, 3)).reshape(batch, seq_len, d)
